# baseline (speedup 1.0000x reference)
import functools

import jax
import jax.numpy as jnp
from jax import lax
from jax.experimental import pallas as pl
from jax.experimental.pallas import tpu as pltpu
from jax.experimental.pallas import tpu_sc as plsc

B = 16384
D = 32
NC = 2
NS = 16
NW = NC * NS
BPW = B // NW
LANES = 16


def _gmf_body(ut, it, ui, ii, out, uidx_v, iidx_v, urows_v, irows_v,
              sem_u, sem_i):
    wid = lax.axis_index("s") * NC + lax.axis_index("c")
    base = wid * BPW
    pltpu.sync_copy(ui.at[pl.ds(base, BPW)], uidx_v)
    pltpu.sync_copy(ii.at[pl.ds(base, BPW)], iidx_v)
    cu = pltpu.async_copy(ut.at[uidx_v], urows_v, sem_u)
    ci = pltpu.async_copy(it.at[iidx_v], irows_v, sem_i)
    cu.wait()
    ci.wait()

    def body(i, carry):
        for j in range(D // LANES):
            sl = pl.ds(j * LANES, LANES)
            urows_v[i, sl] = urows_v[i, sl] * irows_v[i, sl]
        return carry

    lax.fori_loop(0, BPW, body, 0)
    pltpu.sync_copy(urows_v, out.at[pl.ds(base, BPW)])


def kernel(user_table, item_table, user_indices, item_indices):
    mesh = plsc.VectorSubcoreMesh(core_axis_name="c", subcore_axis_name="s")
    k = functools.partial(
        pl.kernel,
        mesh=mesh,
        out_type=jax.ShapeDtypeStruct((B, D), jnp.float32),
        compiler_params=pltpu.CompilerParams(use_tc_tiling_on_sc=False),
        scratch_types=[
            pltpu.VMEM((BPW,), jnp.int32),
            pltpu.VMEM((BPW,), jnp.int32),
            pltpu.VMEM((BPW, D), jnp.float32),
            pltpu.VMEM((BPW, D), jnp.float32),
            pltpu.SemaphoreType.DMA,
            pltpu.SemaphoreType.DMA,
        ],
    )(_gmf_body)
    return k(user_table, item_table, user_indices, item_indices)
